# baseline (speedup 1.0000x reference)
import functools

import jax
import jax.numpy as jnp
from jax import lax
from jax.experimental import pallas as pl
from jax.experimental.pallas import tpu as pltpu
from jax.experimental.pallas import tpu_sc as plsc

N = 10000
E = 320000
D = 128

NC = 2
NS = 16
NW = NC * NS

EPT = E // NW
CH = 80
NCHUNK = EPT // CH
SB = 5
NSUPER = NCHUNK // SB

RPS = 624
RREM = N - NS * RPS
ROWS_PT = 312
LG_REM = N - NW * ROWS_PT

STEADY_START = 2
STEADY_ITERS = 4
PERIOD = 30


def _tc_matmul(x, w):
    def body(x_ref, w_ref, o_ref):
        o_ref[...] = jnp.dot(x_ref[...], w_ref[...],
                             preferred_element_type=jnp.float32)

    return pl.pallas_call(
        body,
        grid=(N // 1000,),
        in_specs=[pl.BlockSpec((1000, D), lambda i: (i, 0)),
                  pl.BlockSpec((D, D), lambda i: (0, 0))],
        out_specs=pl.BlockSpec((1000, D), lambda i: (i, 0)),
        out_shape=jax.ShapeDtypeStruct((N, D), jnp.float32),
    )(x, w)


def _coords(j):
    return (j // SB) % 2, j % SB, j % 3


def _sc_spmm(h, ei4, vals4, perm, re, zblk):
    mesh = plsc.VectorSubcoreMesh(core_axis_name="c", subcore_axis_name="s")

    @functools.partial(
        pl.kernel,
        mesh=mesh,
        out_type=[
            jax.ShapeDtypeStruct((N, D), jnp.float32),
            jax.ShapeDtypeStruct((N, D), jnp.float32),
            jax.ShapeDtypeStruct((N, D), jnp.float32),
        ],
        scratch_types=[
            pltpu.VMEM((SB, CH), jnp.int32),
            pltpu.VMEM((SB, CH), jnp.int32),
            pltpu.VMEM((SB, CH), jnp.int32),
            pltpu.VMEM((SB, CH), jnp.int32),
            pltpu.VMEM((SB + 1, CH), jnp.float32),
            pltpu.VMEM((SB + 1, CH), jnp.float32),
            pltpu.VMEM((CH, D), jnp.float32),
            pltpu.VMEM((CH, D), jnp.float32),
            pltpu.VMEM((CH, D), jnp.float32),
            pltpu.VMEM_SHARED((N, D), jnp.float32),
            pltpu.SemaphoreType.DMA,
            pltpu.SemaphoreType.DMA,
            pltpu.SemaphoreType.DMA,
            pltpu.SemaphoreType.DMA,
            pltpu.SemaphoreType.DMA,
            pltpu.SemaphoreType.DMA,
        ],
    )
    def body(h_hbm, ei_hbm, vals_hbm, perm_hbm, re_hbm, z_hbm,
             part0_hbm, part1_hbm, left_hbm,
             rowA, rowB, colA, colB, valA, valB,
             gb0, gb1, gb2, acc_sh,
             sg0, sg1, sg2, ss0, ss1, ss2):
        c = lax.axis_index("c")
        s = lax.axis_index("s")
        wid = s * NC + c
        rows = (rowA, rowB)
        cols = (colA, colB)
        vls = (valA, valB)
        gbs = (gb0, gb1, gb2)
        sgs = (sg0, sg1, sg2)
        sss = (ss0, ss1, ss2)

        pltpu.sync_copy(z_hbm, acc_sh.at[pl.ds(s * RPS, RPS)])

        @pl.when(s == NS - 1)
        def _():
            pltpu.sync_copy(z_hbm.at[pl.ds(0, RREM)],
                            acc_sh.at[pl.ds(NS * RPS, RREM)])

        for t, lg in enumerate((80, 80, 80, 72)):
            base = wid * ROWS_PT + t * 80
            pltpu.sync_copy(perm_hbm.at[pl.ds(base, lg)],
                            rowA.at[0, pl.ds(0, lg)])
            pltpu.sync_copy(re_hbm.at[rowA.at[0, pl.ds(0, lg)]],
                            gb0.at[pl.ds(0, lg)])
            pltpu.sync_copy(gb0.at[pl.ds(0, lg)],
                            left_hbm.at[pl.ds(base, lg)])

        @pl.when(wid == 0)
        def _():
            pltpu.sync_copy(perm_hbm.at[pl.ds(NW * ROWS_PT, LG_REM)],
                            rowA.at[0, pl.ds(0, LG_REM)])
            pltpu.sync_copy(re_hbm.at[rowA.at[0, pl.ds(0, LG_REM)]],
                            gb0.at[pl.ds(0, LG_REM)])
            pltpu.sync_copy(gb0.at[pl.ds(0, LG_REM)],
                            left_hbm.at[pl.ds(NW * ROWS_PT, LG_REM)])

        plsc.subcore_barrier()

        def load_super(midx, par):
            sb = wid * NSUPER + midx
            pltpu.sync_copy(ei_hbm.at[0, sb], rows[par])
            pltpu.sync_copy(ei_hbm.at[1, sb], cols[par])
            pltpu.sync_copy(vals_hbm.at[sb], vls[par].at[pl.ds(0, SB)])

        def g_idx(par, u):
            return cols[par].at[u]

        def w_idx(par, u):
            return rows[par].at[u]

        def start_gather(par, u, b):
            pltpu.async_copy(h_hbm.at[g_idx(par, u)], gbs[b], sgs[b])

        def wait_gather(par, u, b):
            pltpu.make_async_copy(h_hbm.at[g_idx(par, u)], gbs[b],
                                  sgs[b]).wait()

        def start_scatter(par, u, b):
            pltpu.async_copy(gbs[b], acc_sh.at[w_idx(par, u)], sss[b],
                             add=True)

        def wait_scatter(par, u, b):
            pltpu.make_async_copy(gbs[b], acc_sh.at[w_idx(par, u)],
                                  sss[b]).wait()

        def scale(par, u, b):
            gb = gbs[b]
            vv = vls[par]

            def ebody(ep, carry):
                e0 = ep * 4
                vgrp = vv[u, pl.ds(e0, 16)]
                for k in range(4):
                    v16 = jnp.broadcast_to(vgrp[k], (16,))
                    e = e0 + k
                    for q in range(D // 16):
                        sl = pl.ds(q * 16, 16)
                        gb[e, sl] = gb[e, sl] * v16
                return carry

            lax.fori_loop(0, CH // 4, ebody, 0)

        last_scat = {}

        def emit_chunk(j, i_var=None, with_next=True, with_super=True):
            par, u, b = _coords(j)
            wait_gather(par, u, b)
            if with_next:
                parn, un, bn = _coords(j + 1)
                if with_super and u == 2:
                    m1 = j // SB + 1
                    midx = m1 if i_var is None else 6 * i_var + m1
                    load_super(midx, m1 % 2)
                if bn in last_scat:
                    wait_scatter(*last_scat[bn], bn)
                start_gather(parn, un, bn)
            scale(par, u, b)
            start_scatter(par, u, b)
            last_scat[b] = (par, u)

        load_super(0, 0)
        start_gather(0, 0, 0)
        emit_chunk(0, with_super=False)
        emit_chunk(1, with_super=False)

        def pipe_body(i, carry):
            for off in range(PERIOD):
                emit_chunk(STEADY_START + off, i_var=i)
            return carry

        lax.fori_loop(0, STEADY_ITERS, pipe_body, 0)

        emit_chunk(122, with_super=False)
        emit_chunk(123, with_super=False)
        emit_chunk(124, with_next=False)

        for b in range(3):
            wait_scatter(*last_scat[b], b)

        plsc.subcore_barrier()

        @pl.when(c == 0)
        def _():
            pltpu.sync_copy(acc_sh.at[pl.ds(s * RPS, RPS)],
                            part0_hbm.at[pl.ds(s * RPS, RPS)])

        @pl.when(c == 1)
        def _():
            pltpu.sync_copy(acc_sh.at[pl.ds(s * RPS, RPS)],
                            part1_hbm.at[pl.ds(s * RPS, RPS)])

        @pl.when((s == NS - 1) & (c == 0))
        def _():
            pltpu.sync_copy(acc_sh.at[pl.ds(NS * RPS, RREM)],
                            part0_hbm.at[pl.ds(NS * RPS, RREM)])

        @pl.when((s == NS - 1) & (c == 1))
        def _():
            pltpu.sync_copy(acc_sh.at[pl.ds(NS * RPS, RREM)],
                            part1_hbm.at[pl.ds(NS * RPS, RREM)])

    return body(h, ei4, vals4, perm, re, zblk)


def _tc_final(p0, p1, left, w, b):
    def body(p0_ref, p1_ref, l_ref, w_ref, b_ref, o_ref):
        lft = l_ref[...]
        g = jax.nn.sigmoid(
            jnp.dot(lft, w_ref[...], preferred_element_type=jnp.float32)
            + b_ref[...])
        p = jnp.maximum(p0_ref[...] + p1_ref[...], 0.0)
        o_ref[...] = g * p + (1.0 - g) * lft

    return pl.pallas_call(
        body,
        grid=(N // 1000,),
        in_specs=[pl.BlockSpec((1000, D), lambda i: (i, 0)),
                  pl.BlockSpec((1000, D), lambda i: (i, 0)),
                  pl.BlockSpec((1000, D), lambda i: (i, 0)),
                  pl.BlockSpec((D, D), lambda i: (0, 0)),
                  pl.BlockSpec((1, D), lambda i: (0, 0))],
        out_specs=pl.BlockSpec((1000, D), lambda i: (i, 0)),
        out_shape=jax.ShapeDtypeStruct((N, D), jnp.float32),
    )(p0, p1, left, w, b)


def kernel(right_embed, edge_index, adj_vals, perm, gcnW1,
           highwayWr, highwaybr):
    ei4 = edge_index.astype(jnp.int32).reshape(2, NW * NSUPER, SB, CH)
    vals4 = adj_vals.reshape(NW * NSUPER, SB, CH)
    zblk = jnp.zeros((RPS, D), jnp.float32)

    h = _tc_matmul(right_embed, gcnW1)
    part0, part1, left = _sc_spmm(h, ei4, vals4,
                                  perm.astype(jnp.int32), right_embed, zblk)
    return _tc_final(part0, part1, left, highwayWr,
                     highwaybr.reshape(1, D))


# speedup vs baseline: 1.2141x; 1.2141x over previous
import functools

import jax
import jax.numpy as jnp
from jax import lax
from jax.experimental import pallas as pl
from jax.experimental.pallas import tpu as pltpu
from jax.experimental.pallas import tpu_sc as plsc

N = 10000
E = 320000
D = 128

NC = 2
NS = 16
NW = NC * NS

EPT = E // NW
CH = 80
NCHUNK = EPT // CH
SB = 5
NSUPER = NCHUNK // SB

RPS = 624
RREM = N - NS * RPS
ROWS_PT = 312
LG_REM = N - NW * ROWS_PT

STEADY_START = 2
STEADY_ITERS = 6
PERIOD = 20


def _tc_matmul(x, w):
    def body(x_ref, w_ref, o_ref):
        o_ref[...] = jnp.dot(x_ref[...], w_ref[...],
                             preferred_element_type=jnp.float32)

    return pl.pallas_call(
        body,
        grid=(N // 1000,),
        in_specs=[pl.BlockSpec((1000, D), lambda i: (i, 0)),
                  pl.BlockSpec((D, D), lambda i: (0, 0))],
        out_specs=pl.BlockSpec((1000, D), lambda i: (i, 0)),
        out_shape=jax.ShapeDtypeStruct((N, D), jnp.float32),
    )(x, w)


def _coords(j):
    return (j // SB) % 2, j % SB, j % 4


def _sc_spmm(h, ei4, vals4, perm, re, zblk):
    mesh = plsc.VectorSubcoreMesh(core_axis_name="c", subcore_axis_name="s")

    @functools.partial(
        pl.kernel,
        mesh=mesh,
        out_type=[
            jax.ShapeDtypeStruct((N, D), jnp.float32),
            jax.ShapeDtypeStruct((N, D), jnp.float32),
            jax.ShapeDtypeStruct((N, D), jnp.float32),
        ],
        scratch_types=[
            pltpu.VMEM((SB, CH), jnp.int32),
            pltpu.VMEM((SB, CH), jnp.int32),
            pltpu.VMEM((SB, CH), jnp.int32),
            pltpu.VMEM((SB, CH), jnp.int32),
            pltpu.VMEM((SB + 1, CH), jnp.float32),
            pltpu.VMEM((SB + 1, CH), jnp.float32),
            pltpu.VMEM((CH, D), jnp.float32),
            pltpu.VMEM((CH, D), jnp.float32),
            pltpu.VMEM((CH, D), jnp.float32),
            pltpu.VMEM((CH, D), jnp.float32),
            pltpu.VMEM_SHARED((N, D), jnp.float32),
            pltpu.SemaphoreType.DMA,
            pltpu.SemaphoreType.DMA,
            pltpu.SemaphoreType.DMA,
            pltpu.SemaphoreType.DMA,
            pltpu.SemaphoreType.DMA,
            pltpu.SemaphoreType.DMA,
            pltpu.SemaphoreType.DMA,
            pltpu.SemaphoreType.DMA,
        ],
    )
    def body(h_hbm, ei_hbm, vals_hbm, perm_hbm, re_hbm, z_hbm,
             part0_hbm, part1_hbm, left_hbm,
             rowA, rowB, colA, colB, valA, valB,
             gb0, gb1, gb2, gb3, acc_sh,
             sg0, sg1, sg2, sg3, ss0, ss1, ss2, ss3):
        c = lax.axis_index("c")
        s = lax.axis_index("s")
        wid = s * NC + c
        rows = (rowA, rowB)
        cols = (colA, colB)
        vls = (valA, valB)
        gbs = (gb0, gb1, gb2, gb3)
        sgs = (sg0, sg1, sg2, sg3)
        sss = (ss0, ss1, ss2, ss3)

        pltpu.sync_copy(z_hbm, acc_sh.at[pl.ds(s * RPS, RPS)])

        @pl.when(s == NS - 1)
        def _():
            pltpu.sync_copy(z_hbm.at[pl.ds(0, RREM)],
                            acc_sh.at[pl.ds(NS * RPS, RREM)])

        for t, lg in enumerate((80, 80, 80, 72)):
            base = wid * ROWS_PT + t * 80
            pltpu.sync_copy(perm_hbm.at[pl.ds(base, lg)],
                            rowA.at[0, pl.ds(0, lg)])
            pltpu.sync_copy(re_hbm.at[rowA.at[0, pl.ds(0, lg)]],
                            gb0.at[pl.ds(0, lg)])
            pltpu.sync_copy(gb0.at[pl.ds(0, lg)],
                            left_hbm.at[pl.ds(base, lg)])

        @pl.when(wid == 0)
        def _():
            pltpu.sync_copy(perm_hbm.at[pl.ds(NW * ROWS_PT, LG_REM)],
                            rowA.at[0, pl.ds(0, LG_REM)])
            pltpu.sync_copy(re_hbm.at[rowA.at[0, pl.ds(0, LG_REM)]],
                            gb0.at[pl.ds(0, LG_REM)])
            pltpu.sync_copy(gb0.at[pl.ds(0, LG_REM)],
                            left_hbm.at[pl.ds(NW * ROWS_PT, LG_REM)])

        plsc.subcore_barrier()

        def load_super(midx, par):
            sb = wid * NSUPER + midx
            pltpu.sync_copy(ei_hbm.at[0, sb], rows[par])
            pltpu.sync_copy(ei_hbm.at[1, sb], cols[par])
            pltpu.sync_copy(vals_hbm.at[sb], vls[par].at[pl.ds(0, SB)])

        def g_idx(par, u):
            return cols[par].at[u]

        def w_idx(par, u):
            return rows[par].at[u]

        def start_gather(par, u, b):
            pltpu.async_copy(h_hbm.at[g_idx(par, u)], gbs[b], sgs[b])

        def wait_gather(par, u, b):
            pltpu.make_async_copy(h_hbm.at[g_idx(par, u)], gbs[b],
                                  sgs[b]).wait()

        def start_scatter(par, u, b):
            pltpu.async_copy(gbs[b], acc_sh.at[w_idx(par, u)], sss[b],
                             add=True)

        def wait_scatter(par, u, b):
            pltpu.make_async_copy(gbs[b], acc_sh.at[w_idx(par, u)],
                                  sss[b]).wait()

        def scale(par, u, b):
            gb = gbs[b]
            vv = vls[par]

            def ebody(ep, carry):
                e0 = ep * 4
                vgrp = vv[u, pl.ds(e0, 16)]
                for k in range(4):
                    v16 = jnp.broadcast_to(vgrp[k], (16,))
                    e = e0 + k
                    for q in range(D // 16):
                        sl = pl.ds(q * 16, 16)
                        gb[e, sl] = gb[e, sl] * v16
                return carry

            lax.fori_loop(0, CH // 4, ebody, 0)

        last_scat = {}

        def emit_chunk(j, i_var=None, with_next=True, with_super=True):
            par, u, b = _coords(j)
            wait_gather(par, u, b)
            if with_super and u == 2:
                m1 = j // SB + 1
                midx = m1 if i_var is None else 4 * i_var + m1
                load_super(midx, m1 % 2)
            if with_next:
                parn, un, bn = _coords(j + 2)
                if bn in last_scat:
                    wait_scatter(*last_scat[bn], bn)
                start_gather(parn, un, bn)
            scale(par, u, b)
            start_scatter(par, u, b)
            last_scat[b] = (par, u)

        load_super(0, 0)
        start_gather(0, 0, 0)
        start_gather(0, 1, 1)
        emit_chunk(0, with_super=False)
        emit_chunk(1, with_super=False)

        def pipe_body(i, carry):
            for off in range(PERIOD):
                emit_chunk(STEADY_START + off, i_var=i)
            return carry

        lax.fori_loop(0, STEADY_ITERS, pipe_body, 0)

        emit_chunk(122, with_super=False)
        emit_chunk(123, with_next=False, with_super=False)
        emit_chunk(124, with_next=False)

        for b in range(4):
            wait_scatter(*last_scat[b], b)

        plsc.subcore_barrier()

        @pl.when(c == 0)
        def _():
            pltpu.sync_copy(acc_sh.at[pl.ds(s * RPS, RPS)],
                            part0_hbm.at[pl.ds(s * RPS, RPS)])

        @pl.when(c == 1)
        def _():
            pltpu.sync_copy(acc_sh.at[pl.ds(s * RPS, RPS)],
                            part1_hbm.at[pl.ds(s * RPS, RPS)])

        @pl.when((s == NS - 1) & (c == 0))
        def _():
            pltpu.sync_copy(acc_sh.at[pl.ds(NS * RPS, RREM)],
                            part0_hbm.at[pl.ds(NS * RPS, RREM)])

        @pl.when((s == NS - 1) & (c == 1))
        def _():
            pltpu.sync_copy(acc_sh.at[pl.ds(NS * RPS, RREM)],
                            part1_hbm.at[pl.ds(NS * RPS, RREM)])

    return body(h, ei4, vals4, perm, re, zblk)


def _tc_final(p0, p1, left, w, b):
    def body(p0_ref, p1_ref, l_ref, w_ref, b_ref, o_ref):
        lft = l_ref[...]
        g = jax.nn.sigmoid(
            jnp.dot(lft, w_ref[...], preferred_element_type=jnp.float32)
            + b_ref[...])
        p = jnp.maximum(p0_ref[...] + p1_ref[...], 0.0)
        o_ref[...] = g * p + (1.0 - g) * lft

    return pl.pallas_call(
        body,
        grid=(N // 1000,),
        in_specs=[pl.BlockSpec((1000, D), lambda i: (i, 0)),
                  pl.BlockSpec((1000, D), lambda i: (i, 0)),
                  pl.BlockSpec((1000, D), lambda i: (i, 0)),
                  pl.BlockSpec((D, D), lambda i: (0, 0)),
                  pl.BlockSpec((1, D), lambda i: (0, 0))],
        out_specs=pl.BlockSpec((1000, D), lambda i: (i, 0)),
        out_shape=jax.ShapeDtypeStruct((N, D), jnp.float32),
    )(p0, p1, left, w, b)


def kernel(right_embed, edge_index, adj_vals, perm, gcnW1,
           highwayWr, highwaybr):
    ei4 = edge_index.astype(jnp.int32).reshape(2, NW * NSUPER, SB, CH)
    vals4 = adj_vals.reshape(NW * NSUPER, SB, CH)
    zblk = jnp.zeros((RPS, D), jnp.float32)

    h = _tc_matmul(right_embed, gcnW1)
    part0, part1, left = _sc_spmm(h, ei4, vals4,
                                  perm.astype(jnp.int32), right_embed, zblk)
    return _tc_final(part0, part1, left, highwayWr,
                     highwaybr.reshape(1, D))


# speedup vs baseline: 1.2548x; 1.0335x over previous
import functools

import jax
import jax.numpy as jnp
from jax import lax
from jax.experimental import pallas as pl
from jax.experimental.pallas import tpu as pltpu
from jax.experimental.pallas import tpu_sc as plsc

N = 10000
E = 320000
D = 128

NC = 2
NS = 16
NW = NC * NS

EPT = E // NW
CH = 80
NCHUNK = EPT // CH
SB = 5
NSUPER = NCHUNK // SB

RPS = 624
RREM = N - NS * RPS
ROWS_PT = 312
LG_REM = N - NW * ROWS_PT

STEADY_START = 2
STEADY_ITERS = 6
PERIOD = 20


def _tc_matmul(x, w):
    def body(x_ref, w_ref, o_ref):
        o_ref[...] = jnp.dot(x_ref[...], w_ref[...],
                             preferred_element_type=jnp.float32)

    return pl.pallas_call(
        body,
        grid=(N // 1000,),
        in_specs=[pl.BlockSpec((1000, D), lambda i: (i, 0)),
                  pl.BlockSpec((D, D), lambda i: (0, 0))],
        out_specs=pl.BlockSpec((1000, D), lambda i: (i, 0)),
        out_shape=jax.ShapeDtypeStruct((N, D), jnp.float32),
    )(x, w)


def _coords(j):
    return (j // SB) % 2, j % SB, j % 4


def _sc_spmm(h, ei4, vals4, perm, re, zblk):
    mesh = plsc.VectorSubcoreMesh(core_axis_name="c", subcore_axis_name="s")

    @functools.partial(
        pl.kernel,
        mesh=mesh,
        out_type=[
            jax.ShapeDtypeStruct((N, D), jnp.float32),
            jax.ShapeDtypeStruct((N, D), jnp.float32),
            jax.ShapeDtypeStruct((N, D), jnp.float32),
        ],
        scratch_types=[
            pltpu.VMEM((SB, CH), jnp.int32),
            pltpu.VMEM((SB, CH), jnp.int32),
            pltpu.VMEM((SB, CH), jnp.int32),
            pltpu.VMEM((SB, CH), jnp.int32),
            pltpu.VMEM((SB + 1, CH), jnp.float32),
            pltpu.VMEM((SB + 1, CH), jnp.float32),
            pltpu.VMEM((CH, D), jnp.float32),
            pltpu.VMEM((CH, D), jnp.float32),
            pltpu.VMEM((CH, D), jnp.float32),
            pltpu.VMEM((CH, D), jnp.float32),
            pltpu.VMEM_SHARED((N, D), jnp.float32),
            pltpu.SemaphoreType.DMA,
            pltpu.SemaphoreType.DMA,
            pltpu.SemaphoreType.DMA,
            pltpu.SemaphoreType.DMA,
            pltpu.SemaphoreType.DMA,
            pltpu.SemaphoreType.DMA,
            pltpu.SemaphoreType.DMA,
            pltpu.SemaphoreType.DMA,
            pltpu.SemaphoreType.DMA,
        ],
    )
    def body(h_hbm, ei_hbm, vals_hbm, perm_hbm, re_hbm, z_hbm,
             part0_hbm, part1_hbm, left_hbm,
             rowA, rowB, colA, colB, valA, valB,
             gb0, gb1, gb2, gb3, acc_sh,
             sg0, sg1, sg2, sg3, ss0, ss1, ss2, ss3, sz):
        c = lax.axis_index("c")
        s = lax.axis_index("s")
        wid = s * NC + c
        rows = (rowA, rowB)
        cols = (colA, colB)
        vls = (valA, valB)
        sss = (ss0, ss1, ss2, ss3)

        pltpu.async_copy(z_hbm, acc_sh.at[pl.ds(s * RPS, RPS)], sz)

        @pl.when(s == NS - 1)
        def _():
            pltpu.async_copy(z_hbm.at[pl.ds(0, RREM)],
                             acc_sh.at[pl.ds(NS * RPS, RREM)], sz)

        lgs = (80, 80, 80, 72)
        gbs = (gb0, gb1, gb2, gb3)
        sgs = (sg0, sg1, sg2, sg3)
        for t, lg in enumerate(lgs):
            base = wid * ROWS_PT + t * 80
            pltpu.async_copy(perm_hbm.at[pl.ds(base, lg)],
                             rowA.at[t, pl.ds(0, lg)], sgs[t])
        for t, lg in enumerate(lgs):
            pltpu.make_async_copy(
                perm_hbm.at[pl.ds(wid * ROWS_PT + t * 80, lg)],
                rowA.at[t, pl.ds(0, lg)], sgs[t]).wait()
            pltpu.async_copy(re_hbm.at[rowA.at[t, pl.ds(0, lg)]],
                             gbs[t].at[pl.ds(0, lg)], sgs[t])
        for t, lg in enumerate(lgs):
            base = wid * ROWS_PT + t * 80
            pltpu.make_async_copy(re_hbm.at[rowA.at[t, pl.ds(0, lg)]],
                                  gbs[t].at[pl.ds(0, lg)], sgs[t]).wait()
            pltpu.async_copy(gbs[t].at[pl.ds(0, lg)],
                             left_hbm.at[pl.ds(base, lg)], sgs[t])
        for t, lg in enumerate(lgs):
            base = wid * ROWS_PT + t * 80
            pltpu.make_async_copy(gbs[t].at[pl.ds(0, lg)],
                                  left_hbm.at[pl.ds(base, lg)],
                                  sgs[t]).wait()

        @pl.when(wid == 0)
        def _():
            pltpu.sync_copy(perm_hbm.at[pl.ds(NW * ROWS_PT, LG_REM)],
                            rowA.at[0, pl.ds(0, LG_REM)])
            pltpu.sync_copy(re_hbm.at[rowA.at[0, pl.ds(0, LG_REM)]],
                            gb0.at[pl.ds(0, LG_REM)])
            pltpu.sync_copy(gb0.at[pl.ds(0, LG_REM)],
                            left_hbm.at[pl.ds(NW * ROWS_PT, LG_REM)])

        pltpu.make_async_copy(z_hbm, acc_sh.at[pl.ds(s * RPS, RPS)],
                              sz).wait()

        @pl.when(s == NS - 1)
        def _():
            pltpu.make_async_copy(z_hbm.at[pl.ds(0, RREM)],
                                  acc_sh.at[pl.ds(NS * RPS, RREM)],
                                  sz).wait()

        plsc.subcore_barrier()

        def load_super(midx, par):
            sb = wid * NSUPER + midx
            pltpu.sync_copy(ei_hbm.at[0, sb], rows[par])
            pltpu.sync_copy(ei_hbm.at[1, sb], cols[par])
            pltpu.sync_copy(vals_hbm.at[sb], vls[par].at[pl.ds(0, SB)])

        def g_idx(par, u):
            return cols[par].at[u]

        def w_idx(par, u):
            return rows[par].at[u]

        def start_gather(par, u, b):
            pltpu.async_copy(h_hbm.at[g_idx(par, u)], gbs[b], sgs[b])

        def wait_gather(par, u, b):
            pltpu.make_async_copy(h_hbm.at[g_idx(par, u)], gbs[b],
                                  sgs[b]).wait()

        def start_scatter(par, u, b):
            pltpu.async_copy(gbs[b], acc_sh.at[w_idx(par, u)], sss[b],
                             add=True)

        def wait_scatter(par, u, b):
            pltpu.make_async_copy(gbs[b], acc_sh.at[w_idx(par, u)],
                                  sss[b]).wait()

        def scale(par, u, b):
            gb = gbs[b]
            vv = vls[par]

            def ebody(ep, carry):
                e0 = ep * 4
                vgrp = vv[u, pl.ds(e0, 16)]
                for k in range(4):
                    v16 = jnp.broadcast_to(vgrp[k], (16,))
                    e = e0 + k
                    for q in range(D // 16):
                        sl = pl.ds(q * 16, 16)
                        gb[e, sl] = gb[e, sl] * v16
                return carry

            lax.fori_loop(0, CH // 4, ebody, 0)

        last_scat = {}

        def emit_chunk(j, i_var=None, with_next=True, with_super=True):
            par, u, b = _coords(j)
            wait_gather(par, u, b)
            if with_super and u == 2:
                m1 = j // SB + 1
                midx = m1 if i_var is None else 4 * i_var + m1
                load_super(midx, m1 % 2)
            if with_next:
                parn, un, bn = _coords(j + 2)
                if bn in last_scat:
                    wait_scatter(*last_scat[bn], bn)
                start_gather(parn, un, bn)
            scale(par, u, b)
            start_scatter(par, u, b)
            last_scat[b] = (par, u)

        load_super(0, 0)
        start_gather(0, 0, 0)
        start_gather(0, 1, 1)
        emit_chunk(0, with_super=False)
        emit_chunk(1, with_super=False)

        def pipe_body(i, carry):
            for off in range(PERIOD):
                emit_chunk(STEADY_START + off, i_var=i)
            return carry

        lax.fori_loop(0, STEADY_ITERS, pipe_body, 0)

        emit_chunk(122, with_super=False)
        emit_chunk(123, with_next=False, with_super=False)
        emit_chunk(124, with_next=False)

        for b in range(4):
            wait_scatter(*last_scat[b], b)

        plsc.subcore_barrier()

        @pl.when(c == 0)
        def _():
            pltpu.sync_copy(acc_sh.at[pl.ds(s * RPS, RPS)],
                            part0_hbm.at[pl.ds(s * RPS, RPS)])

        @pl.when(c == 1)
        def _():
            pltpu.sync_copy(acc_sh.at[pl.ds(s * RPS, RPS)],
                            part1_hbm.at[pl.ds(s * RPS, RPS)])

        @pl.when((s == NS - 1) & (c == 0))
        def _():
            pltpu.sync_copy(acc_sh.at[pl.ds(NS * RPS, RREM)],
                            part0_hbm.at[pl.ds(NS * RPS, RREM)])

        @pl.when((s == NS - 1) & (c == 1))
        def _():
            pltpu.sync_copy(acc_sh.at[pl.ds(NS * RPS, RREM)],
                            part1_hbm.at[pl.ds(NS * RPS, RREM)])

    return body(h, ei4, vals4, perm, re, zblk)


def _tc_final(p0, p1, left, w, b):
    def body(p0_ref, p1_ref, l_ref, w_ref, b_ref, o_ref):
        lft = l_ref[...]
        g = jax.nn.sigmoid(
            jnp.dot(lft, w_ref[...], preferred_element_type=jnp.float32)
            + b_ref[...])
        p = jnp.maximum(p0_ref[...] + p1_ref[...], 0.0)
        o_ref[...] = g * p + (1.0 - g) * lft

    return pl.pallas_call(
        body,
        grid=(N // 1000,),
        in_specs=[pl.BlockSpec((1000, D), lambda i: (i, 0)),
                  pl.BlockSpec((1000, D), lambda i: (i, 0)),
                  pl.BlockSpec((1000, D), lambda i: (i, 0)),
                  pl.BlockSpec((D, D), lambda i: (0, 0)),
                  pl.BlockSpec((1, D), lambda i: (0, 0))],
        out_specs=pl.BlockSpec((1000, D), lambda i: (i, 0)),
        out_shape=jax.ShapeDtypeStruct((N, D), jnp.float32),
    )(p0, p1, left, w, b)


def kernel(right_embed, edge_index, adj_vals, perm, gcnW1,
           highwayWr, highwaybr):
    ei4 = edge_index.astype(jnp.int32).reshape(2, NW * NSUPER, SB, CH)
    vals4 = adj_vals.reshape(NW * NSUPER, SB, CH)
    zblk = jnp.zeros((RPS, D), jnp.float32)

    h = _tc_matmul(right_embed, gcnW1)
    part0, part1, left = _sc_spmm(h, ei4, vals4,
                                  perm.astype(jnp.int32), right_embed, zblk)
    return _tc_final(part0, part1, left, highwayWr,
                     highwaybr.reshape(1, D))
